# baseline (speedup 1.0000x reference)
import jax
import jax.numpy as jnp
from jax import lax
from jax.experimental import pallas as pl
from jax.experimental.pallas import tpu as pltpu
from jax.experimental.pallas import tpu_sc as plsc

_C, _T, _HW = 3, 64, 224 * 224
_TS = 16
_HALF = _HW // 2
_NROWS = _C * _T * 2
_SROWS = _C * _TS * 2
_NW = 32
_RPW = _NROWS // _NW


def _body(f_hbm, slow_hbm, fast_hbm, buf):
    w = lax.axis_index("s") * 2 + lax.axis_index("c")
    for j in range(_RPW):
        s = w + j * _NW
        pltpu.sync_copy(f_hbm.at[s], buf)
        pltpu.sync_copy(buf, fast_hbm.at[s])
        c = s // 128
        r = s % 128
        t = r // 2
        h = r % 2
        i = (15 * t + 62) // 63
        is_sel = ((63 * i) // 15) == t

        @pl.when(is_sel)
        def _():
            d = c * 32 + i * 2 + h
            pltpu.sync_copy(buf, slow_hbm.at[d])


_pack = pl.kernel(
    _body,
    out_type=(
        jax.ShapeDtypeStruct((_SROWS, _HALF), jnp.float32),
        jax.ShapeDtypeStruct((_NROWS, _HALF), jnp.float32),
    ),
    mesh=plsc.VectorSubcoreMesh(core_axis_name="c", subcore_axis_name="s"),
    scratch_types=[pltpu.VMEM((_HALF,), jnp.float32)],
)


def kernel(frames):
    f2 = frames.reshape(_NROWS, _HALF)
    slow2, fast2 = _pack(f2)
    return (
        slow2.reshape(_C, _TS, 224, 224),
        fast2.reshape(_C, _T, 224, 224),
    )


# speedup vs baseline: 1.0568x; 1.0568x over previous
import jax
import jax.numpy as jnp
from jax import lax
from jax.experimental import pallas as pl
from jax.experimental.pallas import tpu as pltpu
from jax.experimental.pallas import tpu_sc as plsc

_C, _T, _HW = 3, 64, 224 * 224
_TS = 16
_NR = _C * _T
_SR = _C * _TS
_NW = 32
_RPW = _NR // _NW


def _body(f_hbm, slow_hbm, fast_hbm, b0, b1, l0, l1, s0, s1):
    w = lax.axis_index("s") * 2 + lax.axis_index("c")
    bufs, lsems, ssems = (b0, b1), (l0, l1), (s0, s1)

    def row(j):
        return w + j * _NW

    loads = [None, None]
    pending = [[], []]

    def start_load(j):
        b = j % 2
        loads[b] = pltpu.async_copy(f_hbm.at[row(j)], bufs[b], lsems[b])

    def drain(b):
        for pred in pending[b]:
            if pred is None:
                pltpu.make_async_copy(f_hbm.at[0], bufs[b], ssems[b]).wait()
            else:
                @pl.when(pred)
                def _():
                    pltpu.make_async_copy(f_hbm.at[0], bufs[b], ssems[b]).wait()
        pending[b] = []

    start_load(0)
    for j in range(_RPW):
        b = j % 2
        if j + 1 < _RPW:
            drain(1 - b)
            start_load(j + 1)
        loads[b].wait()
        s = row(j)
        c = s // _T
        t = s % _T
        i = (15 * t + 62) // 63
        is_sel = ((63 * i) // 15) == t
        pltpu.async_copy(bufs[b], fast_hbm.at[s], ssems[b])
        pending[b].append(None)
        d = c * _TS + i

        @pl.when(is_sel)
        def _():
            pltpu.async_copy(bufs[b], slow_hbm.at[d], ssems[b])

        pending[b].append(is_sel)
    drain(0)
    drain(1)


_pack = pl.kernel(
    _body,
    out_type=(
        jax.ShapeDtypeStruct((_SR, _HW), jnp.float32),
        jax.ShapeDtypeStruct((_NR, _HW), jnp.float32),
    ),
    mesh=plsc.VectorSubcoreMesh(core_axis_name="c", subcore_axis_name="s"),
    scratch_types=[
        pltpu.VMEM((_HW,), jnp.float32),
        pltpu.VMEM((_HW,), jnp.float32),
        pltpu.SemaphoreType.DMA,
        pltpu.SemaphoreType.DMA,
        pltpu.SemaphoreType.DMA,
        pltpu.SemaphoreType.DMA,
    ],
)


def kernel(frames):
    f2 = frames.reshape(_NR, _HW)
    slow2, fast2 = _pack(f2)
    return (
        slow2.reshape(_C, _TS, 224, 224),
        fast2.reshape(_C, _T, 224, 224),
    )
